# baseline (speedup 1.0000x reference)
import functools

import jax
import jax.numpy as jnp
from jax import lax
from jax.experimental import pallas as pl
from jax.experimental.pallas import tpu as pltpu


def _mbl_body(conf3_ref, lab3_ref, loc_ref, gt_ref, pos4_ref,
              mask3_ref, acc_ref, *, n_real, r):
    i = pl.program_id(0)
    l3 = 3 * n_real

    x0 = conf3_ref[...]
    x1 = pltpu.roll(x0, l3 - 1, 1)
    x2 = pltpu.roll(x0, l3 - 2, 1)
    m = jnp.maximum(jnp.maximum(x0, x1), x2)
    e0 = jnp.exp(x0 - m)
    e1 = jnp.exp(x1 - m)
    e2 = jnp.exp(x2 - m)
    logs = jnp.log(e0 + e1 + e2)
    bg = logs - (x0 - m)

    lab = lab3_ref[...].astype(jnp.float32)
    col = lax.broadcasted_iota(jnp.int32, x0.shape, 1)
    lane0 = col % 3 == 0
    pos = lane0 & (lab > 0.5)
    isneg = lane0 & (lab < 0.5)
    npos = jnp.sum(jnp.where(pos, 1, 0), axis=1, keepdims=True)
    k = npos * 3
    negcnt = jnp.sum(jnp.where(isneg, 1, 0), axis=1, keepdims=True)
    need = k < negcnt

    mask3_ref[...] = jnp.where(pos | isneg, 1.0, 0.0).astype(jnp.bfloat16)

    @pl.when(jnp.any(need))
    def _slow_path():
        bits = lax.bitcast_convert_type(bg, jnp.uint32)

        def pbody(t, p):
            b = 31 - t
            trial = p | (jnp.uint32(1) << jnp.uint32(b))
            cnt = jnp.sum(jnp.where(isneg & (bits >= trial), 1, 0),
                          axis=1, keepdims=True)
            return jnp.where(cnt >= k, trial, p)

        p = lax.fori_loop(0, 32, pbody, jnp.zeros((r, 1), jnp.uint32))
        gt = isneg & (bits > p)
        g = jnp.sum(jnp.where(gt, 1, 0), axis=1, keepdims=True)
        eq = isneg & (bits == p)
        eneed = k - g

        def tbody(t, tt):
            b = 14 - t
            trial = tt | (1 << b)
            c = jnp.sum(jnp.where(eq & (col < trial), 1, 0),
                        axis=1, keepdims=True)
            return jnp.where(c < eneed, trial, tt)

        tt = lax.fori_loop(0, 15, tbody, jnp.zeros((r, 1), jnp.int32))
        sel = gt | (eq & (col <= tt))
        selneg = (need & sel) | (~need & isneg)
        mask3_ref[...] = jnp.where(pos | selneg, 1.0, 0.0).astype(jnp.bfloat16)

    maskf = mask3_ref[...].astype(jnp.float32)

    ce = jnp.where(lab < 0.5, bg,
                   jnp.where(lab < 1.5, logs - (x1 - m), logs - (x2 - m)))
    w = jnp.where((lab > 0.5) & (lab < 1.5), 2.0, 1.0)
    cls_sum = jnp.sum(maskf * ce * w)

    posf = jnp.where(pos, 1.0, 0.0)
    nposf = jnp.sum(posf)
    mws = jnp.sum(jnp.where(pos, w, 0.0))

    d = loc_ref[...] - gt_ref[...]
    ad = jnp.abs(d)
    s = jnp.where(ad < 1.0, 0.5 * d * d, ad - 0.5)
    p4 = pos4_ref[...].astype(jnp.float32)
    sl1_sum = jnp.sum(s * p4)

    @pl.when(i == 0)
    def _init():
        acc_ref[0] = 0.0
        acc_ref[1] = 0.0
        acc_ref[2] = 0.0
        acc_ref[3] = 0.0

    acc_ref[0] += sl1_sum
    acc_ref[1] += cls_sum
    acc_ref[2] += nposf
    acc_ref[3] += mws


def kernel(confidence, locations, labels, gt_locations):
    B, N, _ = confidence.shape
    R = 16 if B % 16 == 0 else 1

    conf3 = confidence.reshape(B, 3 * N)
    lab3 = jnp.repeat(labels.astype(jnp.bfloat16), 3, axis=1)
    loc_flat = locations.reshape(B, 4 * N)
    gt_flat = gt_locations.reshape(B, 4 * N)
    pos4 = jnp.repeat(jnp.where(labels > 0, jnp.bfloat16(1), jnp.bfloat16(0)),
                      4, axis=1)

    mask3, acc = pl.pallas_call(
        functools.partial(_mbl_body, n_real=N, r=R),
        grid=(B // R,),
        in_specs=[
            pl.BlockSpec((R, 3 * N), lambda i: (i, 0)),
            pl.BlockSpec((R, 3 * N), lambda i: (i, 0)),
            pl.BlockSpec((R, 4 * N), lambda i: (i, 0)),
            pl.BlockSpec((R, 4 * N), lambda i: (i, 0)),
            pl.BlockSpec((R, 4 * N), lambda i: (i, 0)),
        ],
        out_specs=[
            pl.BlockSpec((R, 3 * N), lambda i: (i, 0)),
            pl.BlockSpec(memory_space=pltpu.SMEM),
        ],
        out_shape=[
            jax.ShapeDtypeStruct((B, 3 * N), jnp.bfloat16),
            jax.ShapeDtypeStruct((4,), jnp.float32),
        ],
    )(conf3, lab3, loc_flat, gt_flat, pos4)

    return (acc[0] / acc[2], acc[1] / acc[3], mask3[:, ::3] != 0)


# speedup vs baseline: 2.1605x; 2.1605x over previous
import functools

import jax
import jax.numpy as jnp
from jax import lax
from jax.experimental import pallas as pl
from jax.experimental.pallas import tpu as pltpu


def _mbl_body(conf_ref, lab_ref, loc_ref, gt_ref, pos4_ref,
              mask_ref, acc_ref, nsel_ref, *, n_real, r):
    i = pl.program_id(0)

    c0 = conf_ref[0]
    c1 = conf_ref[1]
    c2 = conf_ref[2]
    m = jnp.maximum(jnp.maximum(c0, c1), c2)
    e0 = jnp.exp(c0 - m)
    e1 = jnp.exp(c1 - m)
    e2 = jnp.exp(c2 - m)
    logs = jnp.log(e0 + e1 + e2)
    bg = logs - (c0 - m)

    lab = lab_ref[...]
    col = lax.broadcasted_iota(jnp.int32, lab.shape, 1)
    valid = col < n_real
    pos = valid & (lab > 0)
    isneg = valid & (lab == 0)
    npos = jnp.sum(jnp.where(pos, 1, 0), axis=1, keepdims=True)
    k = npos * 3
    negcnt = jnp.sum(jnp.where(isneg, 1, 0), axis=1, keepdims=True)
    need = k < negcnt

    nsel_ref[...] = jnp.where(isneg, 1.0, 0.0)

    @pl.when(jnp.any(need))
    def _slow_path():
        bits = lax.bitcast_convert_type(bg, jnp.uint32)

        def pbody(t, p):
            b = 31 - t
            trial = p | (jnp.uint32(1) << jnp.uint32(b))
            cnt = jnp.sum(jnp.where(isneg & (bits >= trial), 1, 0),
                          axis=1, keepdims=True)
            return jnp.where(cnt >= k, trial, p)

        p = lax.fori_loop(0, 32, pbody, jnp.zeros((r, 1), jnp.uint32))
        gt = isneg & (bits > p)
        g = jnp.sum(jnp.where(gt, 1, 0), axis=1, keepdims=True)
        eq = isneg & (bits == p)
        eneed = k - g

        def tbody(t, tt):
            b = 13 - t
            trial = tt | (1 << b)
            c = jnp.sum(jnp.where(eq & (col < trial), 1, 0),
                        axis=1, keepdims=True)
            return jnp.where(c < eneed, trial, tt)

        tt = lax.fori_loop(0, 14, tbody, jnp.zeros((r, 1), jnp.int32))
        sel = gt | (eq & (col <= tt))
        combined = (need & sel) | (~need & isneg)
        nsel_ref[...] = jnp.where(combined, 1.0, 0.0)

    selneg = nsel_ref[...] > 0.5
    mask = pos | selneg
    mask_ref[...] = jnp.where(mask, 1, 0)

    ce = jnp.where(lab == 0, bg,
                   jnp.where(lab == 1, logs - (c1 - m), logs - (c2 - m)))
    w = jnp.where(lab == 1, 2.0, 1.0)
    cls_sum = jnp.sum(jnp.where(mask, ce * w, 0.0))

    posf = jnp.where(pos, 1.0, 0.0)
    nposf = jnp.sum(posf)
    mws = jnp.sum(jnp.where(pos, w, 0.0))

    d = loc_ref[...] - gt_ref[...]
    ad = jnp.abs(d)
    s = jnp.where(ad < 1.0, 0.5 * d * d, ad - 0.5)
    p4 = pos4_ref[...].astype(jnp.float32)
    sl1_sum = jnp.sum(s * p4)

    @pl.when(i == 0)
    def _init():
        acc_ref[0] = 0.0
        acc_ref[1] = 0.0
        acc_ref[2] = 0.0
        acc_ref[3] = 0.0

    acc_ref[0] += sl1_sum
    acc_ref[1] += cls_sum
    acc_ref[2] += nposf
    acc_ref[3] += mws


def kernel(confidence, locations, labels, gt_locations):
    B, N, _ = confidence.shape
    R = 16 if B % 16 == 0 else 1
    NPAD = ((N + 127) // 128) * 128
    pad = NPAD - N

    conf_t = jnp.pad(jnp.moveaxis(confidence, 2, 0), ((0, 0), (0, 0), (0, pad)))
    labp = jnp.pad(labels, ((0, 0), (0, pad)))
    loc_flat = locations.reshape(B, 4 * N)
    gt_flat = gt_locations.reshape(B, 4 * N)
    pos4 = jnp.repeat(jnp.where(labels > 0, jnp.bfloat16(1), jnp.bfloat16(0)),
                      4, axis=1)

    mask_i, acc = pl.pallas_call(
        functools.partial(_mbl_body, n_real=N, r=R),
        grid=(B // R,),
        in_specs=[
            pl.BlockSpec((3, R, NPAD), lambda i: (0, i, 0)),
            pl.BlockSpec((R, NPAD), lambda i: (i, 0)),
            pl.BlockSpec((R, 4 * N), lambda i: (i, 0)),
            pl.BlockSpec((R, 4 * N), lambda i: (i, 0)),
            pl.BlockSpec((R, 4 * N), lambda i: (i, 0)),
        ],
        out_specs=[
            pl.BlockSpec((R, NPAD), lambda i: (i, 0)),
            pl.BlockSpec(memory_space=pltpu.SMEM),
        ],
        out_shape=[
            jax.ShapeDtypeStruct((B, NPAD), jnp.int32),
            jax.ShapeDtypeStruct((4,), jnp.float32),
        ],
        scratch_shapes=[pltpu.VMEM((R, NPAD), jnp.float32)],
    )(conf_t, labp, loc_flat, gt_flat, pos4)

    return (acc[0] / acc[2], acc[1] / acc[3], mask_i[:, :N].astype(bool))


# speedup vs baseline: 2.5394x; 1.1753x over previous
import functools

import jax
import jax.numpy as jnp
from jax import lax
from jax.experimental import pallas as pl
from jax.experimental.pallas import tpu as pltpu


def _mbl_body(conf_ref, lab_ref, diff_ref,
              mask_ref, acc_ref, nsel_ref, *, n_real, r):
    i = pl.program_id(0)

    c0 = conf_ref[0]
    c1 = conf_ref[1]
    c2 = conf_ref[2]
    m = jnp.maximum(jnp.maximum(c0, c1), c2)
    e0 = jnp.exp(c0 - m)
    e1 = jnp.exp(c1 - m)
    e2 = jnp.exp(c2 - m)
    logs = jnp.log(e0 + e1 + e2)
    bg = logs - (c0 - m)

    lab = lab_ref[...]
    col = lax.broadcasted_iota(jnp.int32, lab.shape, 1)
    valid = col < n_real
    pos = valid & (lab > 0)
    isneg = valid & (lab == 0)
    npos = jnp.sum(jnp.where(pos, 1, 0), axis=1, keepdims=True)
    k = npos * 3
    negcnt = jnp.sum(jnp.where(isneg, 1, 0), axis=1, keepdims=True)
    need = k < negcnt

    nsel_ref[...] = jnp.where(isneg, 1.0, 0.0)

    @pl.when(jnp.any(need))
    def _slow_path():
        bits = lax.bitcast_convert_type(bg, jnp.uint32)

        def pbody(t, p):
            b = 31 - t
            trial = p | (jnp.uint32(1) << jnp.uint32(b))
            cnt = jnp.sum(jnp.where(isneg & (bits >= trial), 1, 0),
                          axis=1, keepdims=True)
            return jnp.where(cnt >= k, trial, p)

        p = lax.fori_loop(0, 32, pbody, jnp.zeros((r, 1), jnp.uint32))
        gt = isneg & (bits > p)
        g = jnp.sum(jnp.where(gt, 1, 0), axis=1, keepdims=True)
        eq = isneg & (bits == p)
        eneed = k - g

        def tbody(t, tt):
            b = 13 - t
            trial = tt | (1 << b)
            c = jnp.sum(jnp.where(eq & (col < trial), 1, 0),
                        axis=1, keepdims=True)
            return jnp.where(c < eneed, trial, tt)

        tt = lax.fori_loop(0, 14, tbody, jnp.zeros((r, 1), jnp.int32))
        sel = gt | (eq & (col <= tt))
        combined = (need & sel) | (~need & isneg)
        nsel_ref[...] = jnp.where(combined, 1.0, 0.0)

    selneg = nsel_ref[...] > 0.5
    mask = pos | selneg
    mask_ref[...] = jnp.where(mask, 1, 0)
    maskf = jnp.where(mask, 1.0, 0.0)

    ce = jnp.where(lab == 0, bg,
                   jnp.where(lab == 1, logs - (c1 - m), logs - (c2 - m)))
    w = jnp.where(lab == 1, 2.0, 1.0)
    cls_sum = jnp.sum(ce * w * maskf)

    posf = jnp.where(pos, 1.0, 0.0)

    def sl1(d):
        ad = jnp.abs(d)
        return jnp.where(ad < 1.0, 0.5 * d * d, ad - 0.5)

    sl1s = (sl1(diff_ref[0]) + sl1(diff_ref[1])
            + sl1(diff_ref[2]) + sl1(diff_ref[3]))
    sl1_sum = jnp.sum(sl1s * posf)
    nposf = jnp.sum(posf)
    mws = jnp.sum(w * posf)

    @pl.when(i == 0)
    def _init():
        acc_ref[0] = 0.0
        acc_ref[1] = 0.0
        acc_ref[2] = 0.0
        acc_ref[3] = 0.0

    acc_ref[0] += sl1_sum
    acc_ref[1] += cls_sum
    acc_ref[2] += nposf
    acc_ref[3] += mws


def kernel(confidence, locations, labels, gt_locations):
    B, N, _ = confidence.shape
    R = 8
    NPAD = ((N + 127) // 128) * 128
    pad = NPAD - N

    conf_t = jnp.pad(jnp.moveaxis(confidence, 2, 0), ((0, 0), (0, 0), (0, pad)))
    diff = jnp.pad(jnp.moveaxis(locations - gt_locations, 2, 0),
                   ((0, 0), (0, 0), (0, pad)))
    labp = jnp.pad(labels, ((0, 0), (0, pad)))

    mask_pad, acc = pl.pallas_call(
        functools.partial(_mbl_body, n_real=N, r=R),
        grid=(B // R,),
        in_specs=[
            pl.BlockSpec((3, R, NPAD), lambda i: (0, i, 0)),
            pl.BlockSpec((R, NPAD), lambda i: (i, 0)),
            pl.BlockSpec((4, R, NPAD), lambda i: (0, i, 0)),
        ],
        out_specs=[
            pl.BlockSpec((R, NPAD), lambda i: (i, 0)),
            pl.BlockSpec(memory_space=pltpu.SMEM),
        ],
        out_shape=[
            jax.ShapeDtypeStruct((B, NPAD), jnp.int32),
            jax.ShapeDtypeStruct((4,), jnp.float32),
        ],
        scratch_shapes=[pltpu.VMEM((R, NPAD), jnp.float32)],
    )(conf_t, labp, diff)

    mask = mask_pad[:, :N].astype(bool)
    return (acc[0] / acc[2], acc[1] / acc[3], mask)


# speedup vs baseline: 2.5426x; 1.0013x over previous
import functools

import jax
import jax.numpy as jnp
from jax import lax
from jax.experimental import pallas as pl
from jax.experimental.pallas import tpu as pltpu


def _mbl_body(conf_ref, lab_ref, diff_ref, mask_ref, acc_ref, *, n_real, r):
    i = pl.program_id(0)

    c0 = conf_ref[0]
    c1 = conf_ref[1]
    c2 = conf_ref[2]
    m = jnp.maximum(jnp.maximum(c0, c1), c2)
    e0 = jnp.exp(c0 - m)
    e1 = jnp.exp(c1 - m)
    e2 = jnp.exp(c2 - m)
    logs = jnp.log(e0 + e1 + e2)
    bg = logs - (c0 - m)

    lab = lab_ref[...]
    pos = lab > 0
    isneg = lab == 0
    packed = jnp.sum(jnp.where(pos, 1, 0) + jnp.where(lab == 1, 1 << 16, 0),
                     axis=1, keepdims=True)
    npos = packed & 0xFFFF
    count1 = packed >> 16
    k = npos * 3
    need = k < (n_real - npos)

    mask_ref[...] = jnp.where(pos | isneg, 1, 0)

    @pl.when(jnp.any(need))
    def _slow_path():
        bits = lax.bitcast_convert_type(bg, jnp.uint32)

        def pbody(t, p):
            b = 31 - t
            trial = p | (jnp.uint32(1) << jnp.uint32(b))
            cnt = jnp.sum(jnp.where(isneg & (bits >= trial), 1, 0),
                          axis=1, keepdims=True)
            return jnp.where(cnt >= k, trial, p)

        p = lax.fori_loop(0, 32, pbody, jnp.zeros((r, 1), jnp.uint32))
        gt = isneg & (bits > p)
        g = jnp.sum(jnp.where(gt, 1, 0), axis=1, keepdims=True)
        eq = isneg & (bits == p)
        eneed = k - g
        col = lax.broadcasted_iota(jnp.int32, lab.shape, 1)

        def tbody(t, tt):
            b = 13 - t
            trial = tt | (1 << b)
            c = jnp.sum(jnp.where(eq & (col < trial), 1, 0),
                        axis=1, keepdims=True)
            return jnp.where(c < eneed, trial, tt)

        tt = lax.fori_loop(0, 14, tbody, jnp.zeros((r, 1), jnp.int32))
        sel = gt | (eq & (col <= tt))
        selneg = (need & sel) | (~need & isneg)
        mask_ref[...] = jnp.where(pos | selneg, 1, 0)

    mask = mask_ref[...] > 0

    csel = jnp.where(lab == 1, c1, jnp.where(lab == 2, c2, c0))
    ce = logs - (csel - m)
    w = jnp.where(lab == 1, 2.0, 1.0)
    cls_sum = jnp.sum(jnp.where(mask, ce * w, 0.0))

    posf = jnp.where(pos, 1.0, 0.0)

    def sl1(d):
        ad = jnp.abs(d)
        return jnp.where(ad < 1.0, 0.5 * d * d, ad - 0.5)

    sl1s = (sl1(diff_ref[0]) + sl1(diff_ref[1])
            + sl1(diff_ref[2]) + sl1(diff_ref[3]))
    sl1_sum = jnp.sum(sl1s * posf)

    @pl.when(i == 0)
    def _init():
        acc_ref[0] = 0.0
        acc_ref[1] = 0.0
        acc_ref[2] = 0.0
        acc_ref[3] = 0.0

    acc_ref[0] += sl1_sum
    acc_ref[1] += cls_sum
    acc_ref[2] += jnp.sum(npos).astype(jnp.float32)
    acc_ref[3] += jnp.sum(npos + count1).astype(jnp.float32)


def kernel(confidence, locations, labels, gt_locations):
    B, N, _ = confidence.shape
    R = 8
    NPAD = ((N + 127) // 128) * 128
    pad = NPAD - N

    conf_t = jnp.pad(jnp.moveaxis(confidence, 2, 0), ((0, 0), (0, 0), (0, pad)))
    diff = jnp.pad(jnp.moveaxis(locations - gt_locations, 2, 0),
                   ((0, 0), (0, 0), (0, pad)))
    labp = jnp.pad(labels, ((0, 0), (0, pad)), constant_values=-1)

    mask_pad, acc = pl.pallas_call(
        functools.partial(_mbl_body, n_real=N, r=R),
        grid=(B // R,),
        in_specs=[
            pl.BlockSpec((3, R, NPAD), lambda i: (0, i, 0)),
            pl.BlockSpec((R, NPAD), lambda i: (i, 0)),
            pl.BlockSpec((4, R, NPAD), lambda i: (0, i, 0)),
        ],
        out_specs=[
            pl.BlockSpec((R, NPAD), lambda i: (i, 0)),
            pl.BlockSpec(memory_space=pltpu.SMEM),
        ],
        out_shape=[
            jax.ShapeDtypeStruct((B, NPAD), jnp.int32),
            jax.ShapeDtypeStruct((4,), jnp.float32),
        ],
    )(conf_t, labp, diff)

    mask = mask_pad[:, :N].astype(bool)
    return (acc[0] / acc[2], acc[1] / acc[3], mask)


# speedup vs baseline: 2.5894x; 1.0184x over previous
import functools

import jax
import jax.numpy as jnp
from jax import lax
from jax.experimental import pallas as pl
from jax.experimental.pallas import tpu as pltpu


def _cls_body(conf_ref, lab_ref, mask_ref, acc_ref, *, n_real, r):
    i = pl.program_id(0)

    c0 = conf_ref[0]
    c1 = conf_ref[1]
    c2 = conf_ref[2]
    m = jnp.maximum(jnp.maximum(c0, c1), c2)
    e0 = jnp.exp(c0 - m)
    e1 = jnp.exp(c1 - m)
    e2 = jnp.exp(c2 - m)
    logs = jnp.log(e0 + e1 + e2)
    bg = logs - (c0 - m)

    lab = lab_ref[...]
    pos = lab > 0
    isneg = lab == 0
    packed = jnp.sum(jnp.where(pos, 1, 0) + jnp.where(lab == 1, 1 << 16, 0),
                     axis=1, keepdims=True)
    npos = packed & 0xFFFF
    count1 = packed >> 16
    k = npos * 3
    need = k < (n_real - npos)

    mask_ref[...] = jnp.where(pos | isneg, 1, 0)

    @pl.when(jnp.any(need))
    def _slow_path():
        bits = lax.bitcast_convert_type(bg, jnp.uint32)

        def pbody(t, p):
            b = 31 - t
            trial = p | (jnp.uint32(1) << jnp.uint32(b))
            cnt = jnp.sum(jnp.where(isneg & (bits >= trial), 1, 0),
                          axis=1, keepdims=True)
            return jnp.where(cnt >= k, trial, p)

        p = lax.fori_loop(0, 32, pbody, jnp.zeros((r, 1), jnp.uint32))
        gt = isneg & (bits > p)
        g = jnp.sum(jnp.where(gt, 1, 0), axis=1, keepdims=True)
        eq = isneg & (bits == p)
        eneed = k - g
        col = lax.broadcasted_iota(jnp.int32, lab.shape, 1)

        def tbody(t, tt):
            b = 13 - t
            trial = tt | (1 << b)
            c = jnp.sum(jnp.where(eq & (col < trial), 1, 0),
                        axis=1, keepdims=True)
            return jnp.where(c < eneed, trial, tt)

        tt = lax.fori_loop(0, 14, tbody, jnp.zeros((r, 1), jnp.int32))
        sel = gt | (eq & (col <= tt))
        selneg = (need & sel) | (~need & isneg)
        mask_ref[...] = jnp.where(pos | selneg, 1, 0)

    mask = mask_ref[...] > 0

    csel = jnp.where(lab == 1, c1, jnp.where(lab == 2, c2, c0))
    ce = logs - (csel - m)
    w = jnp.where(lab == 1, 2.0, 1.0)
    cls_sum = jnp.sum(jnp.where(mask, ce * w, 0.0))

    @pl.when(i == 0)
    def _init():
        acc_ref[0] = 0.0
        acc_ref[1] = 0.0
        acc_ref[2] = 0.0

    acc_ref[0] += cls_sum
    acc_ref[1] += jnp.sum(npos).astype(jnp.float32)
    acc_ref[2] += jnp.sum(npos + count1).astype(jnp.float32)


def _loc_body(lab_ref, diff_ref, acc_ref):
    i = pl.program_id(0)
    posf = jnp.where(lab_ref[...] > 0, 1.0, 0.0)

    def sl1(d):
        ad = jnp.abs(d)
        return jnp.where(ad < 1.0, 0.5 * d * d, ad - 0.5)

    sl1s = (sl1(diff_ref[0]) + sl1(diff_ref[1])
            + sl1(diff_ref[2]) + sl1(diff_ref[3]))

    @pl.when(i == 0)
    def _init():
        acc_ref[0] = 0.0

    acc_ref[0] += jnp.sum(sl1s * posf)


def kernel(confidence, locations, labels, gt_locations):
    B, N, _ = confidence.shape
    R = 8
    NPAD = ((N + 127) // 128) * 128
    pad = NPAD - N

    conf_t = jnp.pad(jnp.moveaxis(confidence, 2, 0), ((0, 0), (0, 0), (0, pad)))
    diff = jnp.pad(jnp.moveaxis(locations - gt_locations, 2, 0),
                   ((0, 0), (0, 0), (0, pad)))
    labp = jnp.pad(labels, ((0, 0), (0, pad)), constant_values=-1)

    mask_pad, acc = pl.pallas_call(
        functools.partial(_cls_body, n_real=N, r=R),
        grid=(B // R,),
        in_specs=[
            pl.BlockSpec((3, R, NPAD), lambda i: (0, i, 0)),
            pl.BlockSpec((R, NPAD), lambda i: (i, 0)),
        ],
        out_specs=[
            pl.BlockSpec((R, NPAD), lambda i: (i, 0)),
            pl.BlockSpec(memory_space=pltpu.SMEM),
        ],
        out_shape=[
            jax.ShapeDtypeStruct((B, NPAD), jnp.int32),
            jax.ShapeDtypeStruct((3,), jnp.float32),
        ],
    )(conf_t, labp)

    accl = pl.pallas_call(
        _loc_body,
        grid=(B // R,),
        in_specs=[
            pl.BlockSpec((R, NPAD), lambda i: (i, 0)),
            pl.BlockSpec((4, R, NPAD), lambda i: (0, i, 0)),
        ],
        out_specs=pl.BlockSpec(memory_space=pltpu.SMEM),
        out_shape=jax.ShapeDtypeStruct((1,), jnp.float32),
    )(labp, diff)

    mask = mask_pad[:, :N].astype(bool)
    return (accl[0] / acc[1], acc[0] / acc[2], mask)


# speedup vs baseline: 2.7404x; 1.0583x over previous
import functools

import jax
import jax.numpy as jnp
from jax import lax
from jax.experimental import pallas as pl
from jax.experimental.pallas import tpu as pltpu


def _cls_body(conf_ref, lab_ref, mask_ref, acc_ref, *, n_real, r):
    i = pl.program_id(0)

    c0 = conf_ref[0]
    c1 = conf_ref[1]
    c2 = conf_ref[2]
    m = jnp.maximum(jnp.maximum(c0, c1), c2)
    e0 = jnp.exp(c0 - m)
    e1 = jnp.exp(c1 - m)
    e2 = jnp.exp(c2 - m)
    logs = jnp.log(e0 + e1 + e2)
    bg = logs - (c0 - m)

    lab = lab_ref[...]
    pos = lab > 0
    isneg = lab == 0
    packed = jnp.sum(jnp.where(pos, 1, 0) + jnp.where(lab == 1, 1 << 16, 0),
                     axis=1, keepdims=True)
    npos = packed & 0xFFFF
    count1 = packed >> 16
    k = npos * 3
    need = k < (n_real - npos)

    mask_ref[...] = jnp.where(pos | isneg, 1, 0)

    @pl.when(jnp.any(need))
    def _slow_path():
        bits = lax.bitcast_convert_type(bg, jnp.uint32)

        def pbody(t, p):
            b = 31 - t
            trial = p | (jnp.uint32(1) << jnp.uint32(b))
            cnt = jnp.sum(jnp.where(isneg & (bits >= trial), 1, 0),
                          axis=1, keepdims=True)
            return jnp.where(cnt >= k, trial, p)

        p = lax.fori_loop(0, 32, pbody, jnp.zeros((r, 1), jnp.uint32))
        gt = isneg & (bits > p)
        g = jnp.sum(jnp.where(gt, 1, 0), axis=1, keepdims=True)
        eq = isneg & (bits == p)
        eneed = k - g
        col = lax.broadcasted_iota(jnp.int32, lab.shape, 1)

        def tbody(t, tt):
            b = 13 - t
            trial = tt | (1 << b)
            c = jnp.sum(jnp.where(eq & (col < trial), 1, 0),
                        axis=1, keepdims=True)
            return jnp.where(c < eneed, trial, tt)

        tt = lax.fori_loop(0, 14, tbody, jnp.zeros((r, 1), jnp.int32))
        sel = gt | (eq & (col <= tt))
        selneg = (need & sel) | (~need & isneg)
        mask_ref[...] = jnp.where(pos | selneg, 1, 0)

    mask = mask_ref[...] > 0

    csel = jnp.where(lab == 1, c1, jnp.where(lab == 2, c2, c0))
    ce = logs - (csel - m)
    w = jnp.where(lab == 1, 2.0, 1.0)
    cls_sum = jnp.sum(jnp.where(mask, ce * w, 0.0))

    @pl.when(i == 0)
    def _init():
        acc_ref[0] = 0.0
        acc_ref[1] = 0.0
        acc_ref[2] = 0.0

    acc_ref[0] += cls_sum
    acc_ref[1] += jnp.sum(npos).astype(jnp.float32)
    acc_ref[2] += jnp.sum(npos + count1).astype(jnp.float32)


def _loc_body(lab_ref, diff_ref, acc_ref):
    i = pl.program_id(0)
    posf = jnp.where(lab_ref[...] > 0, 1.0, 0.0)

    def sl1(d):
        ad = jnp.abs(d)
        return jnp.where(ad < 1.0, 0.5 * d * d, ad - 0.5)

    sl1s = (sl1(diff_ref[0]) + sl1(diff_ref[1])
            + sl1(diff_ref[2]) + sl1(diff_ref[3]))

    @pl.when(i == 0)
    def _init():
        acc_ref[0] = 0.0

    acc_ref[0] += jnp.sum(sl1s * posf)


def kernel(confidence, locations, labels, gt_locations):
    B, N, _ = confidence.shape
    R = 16
    NPAD = ((N + 127) // 128) * 128
    pad = NPAD - N

    conf_t = jnp.pad(jnp.moveaxis(confidence, 2, 0), ((0, 0), (0, 0), (0, pad)))
    diff = jnp.pad(jnp.moveaxis(locations - gt_locations, 2, 0),
                   ((0, 0), (0, 0), (0, pad)))
    labp = jnp.pad(labels, ((0, 0), (0, pad)), constant_values=-1)

    mask_pad, acc = pl.pallas_call(
        functools.partial(_cls_body, n_real=N, r=R),
        grid=(B // R,),
        in_specs=[
            pl.BlockSpec((3, R, NPAD), lambda i: (0, i, 0)),
            pl.BlockSpec((R, NPAD), lambda i: (i, 0)),
        ],
        out_specs=[
            pl.BlockSpec((R, NPAD), lambda i: (i, 0)),
            pl.BlockSpec(memory_space=pltpu.SMEM),
        ],
        out_shape=[
            jax.ShapeDtypeStruct((B, NPAD), jnp.int32),
            jax.ShapeDtypeStruct((3,), jnp.float32),
        ],
    )(conf_t, labp)

    accl = pl.pallas_call(
        _loc_body,
        grid=(B // R,),
        in_specs=[
            pl.BlockSpec((R, NPAD), lambda i: (i, 0)),
            pl.BlockSpec((4, R, NPAD), lambda i: (0, i, 0)),
        ],
        out_specs=pl.BlockSpec(memory_space=pltpu.SMEM),
        out_shape=jax.ShapeDtypeStruct((1,), jnp.float32),
    )(labp, diff)

    mask = mask_pad[:, :N].astype(bool)
    return (accl[0] / acc[1], acc[0] / acc[2], mask)
